# baseline (speedup 1.0000x reference)
import functools

import jax
import jax.numpy as jnp
from jax import lax
from jax.experimental import pallas as pl
from jax.experimental.pallas import tpu as pltpu
from jax.experimental.pallas import tpu_sc as plsc

VOCAB = 1000000
EMB = 32
BATCH = 16384
HIST = 50

BLK = 128
PITCH = EMB + 1
NBT = BATCH // BLK
NBLOCKS = HIST * NBT
NBUF = 2


def _softplus16(x):
    t = jnp.exp(-jnp.abs(x))
    s = t / (t + 2.0)
    z = s * s
    p = s * (2.0 + z * (2.0 / 3.0 + z * (2.0 / 5.0 + z * (2.0 / 7.0))))
    return jnp.maximum(x, 0.0) + p


def _make_kernel(num_cores, num_subcores):
    nw = num_cores * num_subcores
    bpw = NBLOCKS // nw
    assert NBLOCKS % nw == 0

    mesh = plsc.VectorSubcoreMesh(core_axis_name="c", subcore_axis_name="s")

    @functools.partial(
        pl.kernel,
        mesh=mesh,
        compiler_params=pltpu.CompilerParams(
            use_tc_tiling_on_sc=False, needs_layout_passes=False),
        out_type=(
            jax.ShapeDtypeStruct((HIST, EMB // 8, NBT, 8, BLK), jnp.float32),
            jax.ShapeDtypeStruct((HIST, EMB // 8, NBT, 8, BLK), jnp.float32),
        ),
        scratch_types=[
            [pltpu.VMEM((BLK,), jnp.int32) for _ in range(NBUF)],
            [pltpu.VMEM((BLK, EMB), jnp.float32) for _ in range(NBUF)],
            [pltpu.VMEM((BLK, EMB), jnp.float32) for _ in range(NBUF)],
            [pltpu.VMEM((EMB, BLK + 1), jnp.float32) for _ in range(NBUF)],
            [pltpu.VMEM((EMB, BLK + 1), jnp.float32) for _ in range(NBUF)],
            [pltpu.SemaphoreType.DMA for _ in range(NBUF)],
            [pltpu.SemaphoreType.DMA for _ in range(NBUF)],
        ],
    )
    def k(ids_t, mu_hbm, sig_hbm, mu_out, sig_out,
          idx_v, mu_rows, sig_rows, mu_t, sig_t, sem_g, sem_w):
        wid = lax.axis_index("s") * num_cores + lax.axis_index("c")
        blk0 = wid * bpw
        iota16 = lax.iota(jnp.int32, 16)
        row_ids = [iota16 + (g * 16) for g in range(BLK // 16)]

        def hb(kk):
            blk = blk0 + kk
            return blk // NBT, blk % NBT

        def stage_and_fire(kk, b):
            h, bt = hb(kk)
            pltpu.sync_copy(ids_t.at[h, pl.ds(bt * BLK, BLK)], idx_v[b])
            pltpu.async_copy(mu_hbm.at[idx_v[b]], mu_rows[b], sem_g[b])
            pltpu.async_copy(sig_hbm.at[idx_v[b]], sig_rows[b], sem_g[b])

        def wait_gathers(b):
            pltpu.make_async_copy(mu_hbm.at[idx_v[b]], mu_rows[b], sem_g[b]).wait()
            pltpu.make_async_copy(sig_hbm.at[idx_v[b]], sig_rows[b], sem_g[b]).wait()

        def compute(b):
            @plsc.parallel_loop(0, BLK, unroll=8)
            def row_body(r):
                r_splat = jnp.full((16,), r, jnp.int32)
                for half in range(2):
                    cs = pl.ds(half * 16, 16)
                    ji = iota16 + (half * 16)
                    plsc.store_scatter(mu_t[b], [ji, r_splat], mu_rows[b][r, cs])
                    plsc.store_scatter(sig_t[b], [ji, r_splat],
                                       _softplus16(sig_rows[b][r, cs]))

        def fire_writeback(kk, b):
            h, bt = hb(kk)
            for jt in range(EMB // 8):
                pltpu.async_copy(mu_t[b].at[pl.ds(jt * 8, 8), pl.ds(0, BLK)],
                                 mu_out.at[h, jt, bt], sem_w[b])
                pltpu.async_copy(sig_t[b].at[pl.ds(jt * 8, 8), pl.ds(0, BLK)],
                                 sig_out.at[h, jt, bt], sem_w[b])

        def wait_writeback(b):
            for jt in range(EMB // 8):
                pltpu.make_async_copy(mu_t[b].at[pl.ds(jt * 8, 8), pl.ds(0, BLK)],
                                      mu_out.at[0, jt, 0], sem_w[b]).wait()
                pltpu.make_async_copy(sig_t[b].at[pl.ds(jt * 8, 8), pl.ds(0, BLK)],
                                      sig_out.at[0, jt, 0], sem_w[b]).wait()

        stage_and_fire(0, 0)

        def outer(g, carry):
            for b in range(NBUF):
                kk = NBUF * g + b
                nb = 1 - b

                @pl.when(kk >= 1)
                def _():
                    wait_writeback(nb)

                @pl.when(kk + 1 < bpw)
                def _():
                    stage_and_fire(kk + 1, nb)

                wait_gathers(b)
                compute(b)
                fire_writeback(kk, b)
            return carry

        lax.fori_loop(0, bpw // NBUF, outer, 0)
        wait_writeback((bpw - 1) % NBUF)

    return k


@jax.jit
def kernel(input_ids, mu_table, sigma_table):
    info = plsc.get_sparse_core_info()
    k = _make_kernel(info.num_cores, info.num_subcores)
    ids_t = jnp.transpose(input_ids.astype(jnp.int32))
    mu5, sig5 = k(ids_t, mu_table, sigma_table)
    mu = mu5.transpose(2, 4, 0, 1, 3).reshape(BATCH, HIST, EMB)
    sig = sig5.transpose(2, 4, 0, 1, 3).reshape(BATCH, HIST, EMB)
    return (mu, sig)


# speedup vs baseline: 1.0717x; 1.0717x over previous
import functools

import jax
import jax.numpy as jnp
from jax import lax
from jax.experimental import pallas as pl
from jax.experimental.pallas import tpu as pltpu
from jax.experimental.pallas import tpu_sc as plsc

VOCAB = 1000000
EMB = 32
BATCH = 16384
HIST = 50

BLK = 128
PITCH = EMB + 1
NBT = BATCH // BLK
NBLOCKS = HIST * NBT
NBUF = 4
LA = 2


def _softplus16(x):
    t = jnp.exp(-jnp.abs(x))
    s = t / (t + 2.0)
    z = s * s
    p = s * (2.0 + z * (2.0 / 3.0 + z * (2.0 / 5.0 + z * (2.0 / 7.0))))
    return jnp.maximum(x, 0.0) + p


def _make_kernel(num_cores, num_subcores):
    nw = num_cores * num_subcores
    bpw = NBLOCKS // nw
    assert NBLOCKS % nw == 0

    mesh = plsc.VectorSubcoreMesh(core_axis_name="c", subcore_axis_name="s")

    @functools.partial(
        pl.kernel,
        mesh=mesh,
        compiler_params=pltpu.CompilerParams(
            use_tc_tiling_on_sc=False, needs_layout_passes=False),
        out_type=(
            jax.ShapeDtypeStruct((HIST, EMB // 8, NBT, 8, BLK), jnp.float32),
            jax.ShapeDtypeStruct((HIST, EMB // 8, NBT, 8, BLK), jnp.float32),
        ),
        scratch_types=[
            [pltpu.VMEM((BLK,), jnp.int32) for _ in range(NBUF)],
            [pltpu.VMEM((BLK, EMB), jnp.float32) for _ in range(NBUF)],
            [pltpu.VMEM((BLK, EMB), jnp.float32) for _ in range(NBUF)],
            [pltpu.VMEM((EMB, BLK + 1), jnp.float32) for _ in range(NBUF)],
            [pltpu.VMEM((EMB, BLK + 1), jnp.float32) for _ in range(NBUF)],
            [pltpu.SemaphoreType.DMA for _ in range(NBUF)],
            [pltpu.SemaphoreType.DMA for _ in range(NBUF)],
        ],
    )
    def k(ids_t, mu_hbm, sig_hbm, mu_out, sig_out,
          idx_v, mu_rows, sig_rows, mu_t, sig_t, sem_g, sem_w):
        wid = lax.axis_index("s") * num_cores + lax.axis_index("c")
        blk0 = wid * bpw
        iota16 = lax.iota(jnp.int32, 16)
        row_ids = [iota16 + (g * 16) for g in range(BLK // 16)]

        def hb(kk):
            blk = blk0 + kk
            return blk // NBT, blk % NBT

        def stage_and_fire(kk, b):
            h, bt = hb(kk)
            pltpu.sync_copy(ids_t.at[h, pl.ds(bt * BLK, BLK)], idx_v[b])
            pltpu.async_copy(mu_hbm.at[idx_v[b]], mu_rows[b], sem_g[b])
            pltpu.async_copy(sig_hbm.at[idx_v[b]], sig_rows[b], sem_g[b])

        def wait_gathers(b):
            pltpu.make_async_copy(mu_hbm.at[idx_v[b]], mu_rows[b], sem_g[b]).wait()
            pltpu.make_async_copy(sig_hbm.at[idx_v[b]], sig_rows[b], sem_g[b]).wait()

        def compute(b):
            @plsc.parallel_loop(0, BLK, unroll=4)
            def row_body(r):
                r_splat = jnp.full((16,), r, jnp.int32)
                for half in range(2):
                    cs = pl.ds(half * 16, 16)
                    ji = iota16 + (half * 16)
                    plsc.store_scatter(mu_t[b], [ji, r_splat], mu_rows[b][r, cs])
                    plsc.store_scatter(sig_t[b], [ji, r_splat],
                                       _softplus16(sig_rows[b][r, cs]))

        def fire_writeback(kk, b):
            h, bt = hb(kk)
            for jt in range(EMB // 8):
                pltpu.async_copy(mu_t[b].at[pl.ds(jt * 8, 8), pl.ds(0, BLK)],
                                 mu_out.at[h, jt, bt], sem_w[b])
                pltpu.async_copy(sig_t[b].at[pl.ds(jt * 8, 8), pl.ds(0, BLK)],
                                 sig_out.at[h, jt, bt], sem_w[b])

        def wait_writeback(b):
            for jt in range(EMB // 8):
                pltpu.make_async_copy(mu_t[b].at[pl.ds(jt * 8, 8), pl.ds(0, BLK)],
                                      mu_out.at[0, jt, 0], sem_w[b]).wait()
                pltpu.make_async_copy(sig_t[b].at[pl.ds(jt * 8, 8), pl.ds(0, BLK)],
                                      sig_out.at[0, jt, 0], sem_w[b]).wait()

        for j in range(LA):
            stage_and_fire(j, j)

        def outer(g, carry):
            for b in range(NBUF):
                kk = NBUF * g + b
                nb = (b + LA) % NBUF

                @pl.when(kk >= NBUF - LA)
                def _():
                    wait_writeback(nb)

                @pl.when(kk + LA < bpw)
                def _():
                    stage_and_fire(kk + LA, nb)

                wait_gathers(b)
                compute(b)
                fire_writeback(kk, b)
            return carry

        lax.fori_loop(0, bpw // NBUF, outer, 0)
        for j in range(LA):
            wait_writeback((bpw - LA + j) % NBUF)

    return k


@jax.jit
def kernel(input_ids, mu_table, sigma_table):
    info = plsc.get_sparse_core_info()
    k = _make_kernel(info.num_cores, info.num_subcores)
    ids_t = jnp.transpose(input_ids.astype(jnp.int32))
    mu5, sig5 = k(ids_t, mu_table, sigma_table)
    mu = mu5.transpose(2, 4, 0, 1, 3).reshape(BATCH, HIST, EMB)
    sig = sig5.transpose(2, 4, 0, 1, 3).reshape(BATCH, HIST, EMB)
    return (mu, sig)


# speedup vs baseline: 1.0870x; 1.0143x over previous
import functools

import jax
import jax.numpy as jnp
from jax import lax
from jax.experimental import pallas as pl
from jax.experimental.pallas import tpu as pltpu
from jax.experimental.pallas import tpu_sc as plsc

VOCAB = 1000000
EMB = 32
BATCH = 16384
HIST = 50

BLK = 128
NBT = BATCH // BLK
NBLOCKS = HIST * NBT
NBUF = 4
LA = 2


def _softplus16(x):
    t = jnp.exp(-jnp.abs(x))
    s = t / (t + 2.0)
    z = s * s
    p = s * (2.0 + z * (2.0 / 3.0 + z * (2.0 / 5.0 + z * (2.0 / 7.0))))
    return jnp.maximum(x, 0.0) + p


def _make_kernel(num_cores, num_subcores, apply_softplus):
    nw = num_cores * num_subcores
    bpw = NBLOCKS // nw
    assert NBLOCKS % nw == 0

    mesh = plsc.VectorSubcoreMesh(core_axis_name="c", subcore_axis_name="s")

    @functools.partial(
        pl.kernel,
        mesh=mesh,
        compiler_params=pltpu.CompilerParams(
            use_tc_tiling_on_sc=False, needs_layout_passes=False),
        out_type=jax.ShapeDtypeStruct((HIST, EMB // 8, NBT, 8, BLK),
                                      jnp.float32),
        scratch_types=[
            [pltpu.VMEM((BLK,), jnp.int32) for _ in range(NBUF)],
            [pltpu.VMEM((BLK, EMB), jnp.float32) for _ in range(NBUF)],
            [pltpu.VMEM((EMB, BLK + 1), jnp.float32) for _ in range(NBUF)],
            [pltpu.SemaphoreType.DMA for _ in range(NBUF)],
            [pltpu.SemaphoreType.DMA for _ in range(NBUF)],
        ],
    )
    def k(ids_t, tbl_hbm, out, idx_v, rows, tbuf, sem_g, sem_w):
        wid = lax.axis_index("s") * num_cores + lax.axis_index("c")
        blk0 = wid * bpw
        iota16 = lax.iota(jnp.int32, 16)

        def hb(kk):
            blk = blk0 + kk
            return blk // NBT, blk % NBT

        def stage_and_fire(kk, b):
            h, bt = hb(kk)
            pltpu.sync_copy(ids_t.at[h, pl.ds(bt * BLK, BLK)], idx_v[b])
            pltpu.async_copy(tbl_hbm.at[idx_v[b]], rows[b], sem_g[b])

        def wait_gathers(b):
            pltpu.make_async_copy(tbl_hbm.at[idx_v[b]], rows[b], sem_g[b]).wait()

        def compute(b):
            @plsc.parallel_loop(0, BLK, unroll=4)
            def row_body(r):
                r_splat = jnp.full((16,), r, jnp.int32)
                for half in range(2):
                    cs = pl.ds(half * 16, 16)
                    ji = iota16 + (half * 16)
                    x = rows[b][r, cs]
                    if apply_softplus:
                        x = _softplus16(x)
                    plsc.store_scatter(tbuf[b], [ji, r_splat], x)

        def fire_writeback(kk, b):
            h, bt = hb(kk)
            for jt in range(EMB // 8):
                pltpu.async_copy(tbuf[b].at[pl.ds(jt * 8, 8), pl.ds(0, BLK)],
                                 out.at[h, jt, bt], sem_w[b])

        def wait_writeback(b):
            for jt in range(EMB // 8):
                pltpu.make_async_copy(tbuf[b].at[pl.ds(jt * 8, 8), pl.ds(0, BLK)],
                                      out.at[0, jt, 0], sem_w[b]).wait()

        for j in range(LA):
            stage_and_fire(j, j)

        def outer(g, carry):
            for b in range(NBUF):
                kk = NBUF * g + b
                nb = (b + LA) % NBUF

                @pl.when(kk >= NBUF - LA)
                def _():
                    wait_writeback(nb)

                @pl.when(kk + LA < bpw)
                def _():
                    stage_and_fire(kk + LA, nb)

                wait_gathers(b)
                compute(b)
                fire_writeback(kk, b)
            return carry

        lax.fori_loop(0, bpw // NBUF, outer, 0)
        for j in range(LA):
            wait_writeback((bpw - LA + j) % NBUF)

    return k


@jax.jit
def kernel(input_ids, mu_table, sigma_table):
    info = plsc.get_sparse_core_info()
    k_mu = _make_kernel(info.num_cores, info.num_subcores, False)
    k_sig = _make_kernel(info.num_cores, info.num_subcores, True)
    ids_t = jnp.transpose(input_ids.astype(jnp.int32))
    mu5 = k_mu(ids_t, mu_table)
    sig5 = k_sig(ids_t, sigma_table)
    mu = mu5.transpose(2, 4, 0, 1, 3).reshape(BATCH, HIST, EMB)
    sig = sig5.transpose(2, 4, 0, 1, 3).reshape(BATCH, HIST, EMB)
    return (mu, sig)
